# baseline (speedup 1.0000x reference)
import functools

import jax
import jax.numpy as jnp
from jax import lax
from jax.experimental import pallas as pl
from jax.experimental.pallas import tpu as pltpu
from jax.experimental.pallas import tpu_sc as plsc

ROWS = 128
COLS = 32768
K = 64
NC = 2
NS = 16
NW = NC * NS
RPW = ROWS // NW
CPL = COLS // 16
SIGN = -(2 ** 31)
M31 = 0x7FFFFFFF


def _shrl(x, n):
    return lax.shift_right_logical(x, lax.full_like(x, n))


def _ukey(v):
    b = plsc.bitcast(v, jnp.int32)
    m = lax.shift_right_arithmetic(b, lax.full_like(b, 31))
    return b ^ (m | jnp.int32(SIGN))


def _byte(u, s):
    if s == 24:
        return _shrl(u, 24)
    return jnp.bitwise_and(_shrl(u, s), jnp.int32(255))


def _zero_hist(hist_v):
    zero16i = jnp.zeros((16,), jnp.int32)

    @plsc.parallel_loop(0, 16 * 256, step=16, unroll=8)
    def _zb(i):
        hist_v[pl.ds(i, 16)] = zero16i


def _collapse(hist_v, histc_v, lane):
    @plsc.parallel_loop(0, 256, step=16)
    def _cg(gbase):
        a = hist_v[pl.ds(gbase, 16)]
        for l in range(1, 16):
            a = a + hist_v[pl.ds(l * 256 + gbase, 16)]
        plsc.store_scatter(histc_v, [lane * 16 + (gbase >> 4)], a)


def _find_digit(histc_v, kk, lane):
    gtot = histc_v[pl.ds(0, 16)]
    for i in range(1, 16):
        gtot = gtot + histc_v[pl.ds(i * 16, 16)]
    grev = lax.rev(gtot, (0,))
    gcs = lax.cumsum(grev, axis=0)
    gidx = jnp.max(plsc.all_reduce_ffs(gcs >= kk))
    g_star = 15 - gidx
    above_g = jnp.sum(jnp.where(lane == gidx, gcs - grev, 0))

    h = plsc.load_gather(histc_v, [lane * 16 + g_star])
    rev = lax.rev(h, (0,))
    cs = lax.cumsum(rev, axis=0)
    idx = jnp.max(plsc.all_reduce_ffs((above_g + cs) >= kk))
    digit = g_star * 16 + 15 - idx
    above_d = above_g + jnp.sum(jnp.where(lane == idx, cs - rev, 0))
    return digit, kk - above_d


def _row_phase_a(row_v, candA_v, hist_v, histc_v, lane, laneoff,
                 lanecpl):
    ones_i = jnp.ones((16,), jnp.int32)
    mask0 = lane >= 1

    _zero_hist(hist_v)
    v0 = row_v[pl.ds(0, 16)]
    u0 = _ukey(v0)
    plsc.addupdate_scatter(
        hist_v, [_byte(u0, 24) + laneoff], ones_i, mask=mask0)

    @plsc.parallel_loop(16, COLS, step=16, unroll=8)
    def _p1(i):
        v = row_v[pl.ds(i, 16)]
        plsc.addupdate_scatter(
            hist_v, [_byte(_ukey(v), 24) + laneoff], ones_i)

    _collapse(hist_v, histc_v, lane)
    b1, rem = _find_digit(histc_v, jnp.int32(K), lane)

    _zero_hist(hist_v)
    hi0 = _byte(u0, 24)
    gt0 = jnp.logical_and(hi0 > b1, mask0)
    eq0 = jnp.logical_and(hi0 == b1, mask0)
    s0 = jnp.where(gt0, jnp.exp(v0), jnp.float32(0.0))
    plsc.addupdate_scatter(hist_v, [_byte(u0, 16) + laneoff], ones_i,
                           mask=eq0)
    plsc.store_scatter(candA_v, [lanecpl], v0, mask=eq0)
    off0 = jnp.where(eq0, 1, 0)

    @plsc.parallel_loop(16, COLS, step=16, unroll=8, carry=(s0, off0))
    def _p2(i, carry):
        s, off = carry
        v = row_v[pl.ds(i, 16)]
        u = _ukey(v)
        hi = _byte(u, 24)
        gt = hi > b1
        eq = hi == b1
        s = s + jnp.where(gt, jnp.exp(v), jnp.float32(0.0))
        plsc.addupdate_scatter(hist_v, [_byte(u, 16) + laneoff], ones_i,
                               mask=eq)
        plsc.store_scatter(candA_v, [lanecpl + off], v, mask=eq)
        return s, off + jnp.where(eq, 1, 0)

    s_gt, n1 = _p2
    return b1, rem, s_gt, n1


def _row_phase_b(state, candA_v, candB_v, hist_v, histc_v, lane, laneoff,
                 lanecpl):
    b1, rem, s_gt_v0, n1 = state
    ones_i = jnp.ones((16,), jnp.int32)
    zf = jnp.zeros((16,), jnp.float32)
    zi = jnp.zeros((16,), jnp.int32)
    _collapse(hist_v, histc_v, lane)
    b2, rem = _find_digit(histc_v, rem, lane)
    s_gt = s_gt_v0

    def cand_pass(src_v, dst_v, n_src, s_byte, bsel):
        _zero_hist(hist_v)
        trips = jnp.max(n_src)

        def body(j, carry):
            s, off = carry
            v = plsc.load_gather(src_v, [lanecpl + j])
            valid = j < n_src
            u = _ukey(v)
            d = _byte(u, s_byte)
            gt = jnp.logical_and(valid, d > bsel)
            eq = jnp.logical_and(valid, d == bsel)
            s = s + jnp.where(gt, jnp.exp(v), jnp.float32(0.0))
            plsc.addupdate_scatter(hist_v, [_byte(u, s_byte - 8) + laneoff],
                                   ones_i, mask=eq)
            plsc.store_scatter(dst_v, [lanecpl + off], v, mask=eq)
            return s, off + jnp.where(eq, 1, 0)

        s_add, n_dst = lax.fori_loop(0, trips, body, (zf, zi))
        _collapse(hist_v, histc_v, lane)
        return s_add, n_dst

    s3, n2 = cand_pass(candA_v, candB_v, n1, 16, b2)
    s_gt = s_gt + s3
    b3, rem = _find_digit(histc_v, rem, lane)

    s4, n3 = cand_pass(candB_v, candA_v, n2, 8, b3)
    s_gt = s_gt + s4
    b4, rem = _find_digit(histc_v, rem, lane)

    trips3 = jnp.max(n3)

    def p5(j, s):
        v = plsc.load_gather(candA_v, [lanecpl + j])
        valid = j < n3
        d = _byte(_ukey(v), 0)
        gt = jnp.logical_and(valid, d > b4)
        return s + jnp.where(gt, jnp.exp(v), jnp.float32(0.0))

    s_gt = s_gt + lax.fori_loop(0, trips3, p5, zf)
    sum_gt = jnp.sum(s_gt)

    t_u = jnp.bitwise_or(
        jnp.left_shift(
            jnp.bitwise_or(
                jnp.left_shift(
                    jnp.bitwise_or(jnp.left_shift(b1, 8), b2), 8),
                b3), 8),
        b4)
    tvec = jnp.full((16,), t_u, jnp.int32)
    bt = jnp.where(tvec < 0, tvec ^ jnp.int32(SIGN),
                   jnp.bitwise_xor(tvec, jnp.int32(-1)))
    exp_t = jnp.max(jnp.exp(plsc.bitcast(bt, jnp.float32)))

    return sum_gt + rem.astype(jnp.float32) * exp_t


def _sc_body(y_hbm, out_hbm, row_v, candA_v, candB_v, hist_v, histc_v,
             ng_v, sem):
    wid = lax.axis_index("s") * NC + lax.axis_index("c")
    lane = lax.iota(jnp.int32, 16)
    laneoff = lane * 256
    lanecpl = lane * CPL

    ngvec0 = jnp.ones((16,), jnp.float32)
    pltpu.sync_copy(y_hbm.at[wid * RPW], row_v)

    def rbody(j, ngvec):
        state = _row_phase_a(row_v, candA_v, hist_v, histc_v, lane,
                             laneoff, lanecpl)
        nxt = wid * RPW + jnp.minimum(j + 1, RPW - 1)
        cp = pltpu.async_copy(y_hbm.at[nxt], row_v, sem)
        ng = _row_phase_b(state, candA_v, candB_v, hist_v, histc_v, lane,
                          laneoff, lanecpl)
        cp.wait()
        return jnp.where(lane == j, ng, ngvec)

    ngvec = lax.fori_loop(0, RPW, rbody, ngvec0)
    ng_v[...] = ngvec
    pltpu.sync_copy(ng_v, out_hbm.at[wid])


def _totals_kernel(y_ref, tot_ref):
    i = pl.program_id(0)
    x = y_ref[...]
    col = lax.broadcasted_iota(jnp.int32, x.shape, 1)
    e = jnp.exp(x)
    e = jnp.where(jnp.logical_and(i == 0, col == 0), 0.0, e)
    part = jnp.sum(e, axis=1, keepdims=True)

    @pl.when(i == 0)
    def _init():
        tot_ref[...] = jnp.zeros_like(tot_ref)

    tot_ref[...] += part


def _finish_kernel(tot_ref, top_ref, y_ref, out_ref):
    ng = tot_ref[...] - top_ref[...]
    y = y_ref[...]
    col = lax.broadcasted_iota(jnp.int32, y.shape, 1)
    pos_sum = jnp.sum(jnp.where(col == 0, y, 0.0))
    out_ref[0, 0] = (jnp.sum(jnp.log(ng)) - pos_sum) * (1.0 / ROWS)


@jax.jit
def kernel(y_pred):
    mesh = plsc.VectorSubcoreMesh(core_axis_name="c", subcore_axis_name="s")
    sc = pl.kernel(
        _sc_body,
        out_type=jax.ShapeDtypeStruct((NW, 16), jnp.float32),
        mesh=mesh,
        compiler_params=pltpu.CompilerParams(needs_layout_passes=False),
        scratch_types=[
            pltpu.VMEM((COLS,), jnp.float32),
            pltpu.VMEM((COLS,), jnp.float32),
            pltpu.VMEM((COLS,), jnp.float32),
            pltpu.VMEM((16 * 256,), jnp.int32),
            pltpu.VMEM((256,), jnp.int32),
            pltpu.VMEM((16,), jnp.float32),
            pltpu.SemaphoreType.DMA,
        ],
    )
    top = sc(y_pred)

    cblk = 4096
    totals = pl.pallas_call(
        _totals_kernel,
        grid=(COLS // cblk,),
        in_specs=[pl.BlockSpec((ROWS, cblk), lambda i: (0, i))],
        out_specs=pl.BlockSpec((ROWS, 1), lambda i: (0, 0)),
        out_shape=jax.ShapeDtypeStruct((ROWS, 1), jnp.float32),
    )(y_pred)

    top128 = top[:, :RPW].reshape(ROWS, 1)

    out = pl.pallas_call(
        _finish_kernel,
        grid=(1,),
        in_specs=[
            pl.BlockSpec((ROWS, 1), lambda i: (0, 0)),
            pl.BlockSpec((ROWS, 1), lambda i: (0, 0)),
            pl.BlockSpec((ROWS, 128), lambda i: (0, 0)),
        ],
        out_specs=pl.BlockSpec((1, 1), lambda i: (0, 0),
                               memory_space=pltpu.SMEM),
        out_shape=jax.ShapeDtypeStruct((1, 1), jnp.float32),
    )(totals, top128, y_pred)
    return out[0, 0]


# speedup vs baseline: 1.0959x; 1.0959x over previous
import functools

import jax
import jax.numpy as jnp
from jax import lax
from jax.experimental import pallas as pl
from jax.experimental.pallas import tpu as pltpu
from jax.experimental.pallas import tpu_sc as plsc

ROWS = 128
COLS = 32768
K = 64
NC = 2
NS = 16
NW = NC * NS
RPW = ROWS // NW
CPL = COLS // 16
SIGN = -(2 ** 31)
M31 = 0x7FFFFFFF


def _shrl(x, n):
    return lax.shift_right_logical(x, lax.full_like(x, n))


def _ukey(v):
    b = plsc.bitcast(v, jnp.int32)
    m = lax.shift_right_arithmetic(b, lax.full_like(b, 31))
    return b ^ (m | jnp.int32(SIGN))


def _byte(u, s):
    if s == 24:
        return _shrl(u, 24)
    return jnp.bitwise_and(_shrl(u, s), jnp.int32(255))


def _zero_hist(hist_v):
    zero16i = jnp.zeros((16,), jnp.int32)

    @plsc.parallel_loop(0, 16 * 256, step=16, unroll=8)
    def _zb(i):
        hist_v[pl.ds(i, 16)] = zero16i


def _collapse(hist_v, histc_v, lane):
    @plsc.parallel_loop(0, 256, step=16)
    def _cg(gbase):
        a = hist_v[pl.ds(gbase, 16)]
        for l in range(1, 16):
            a = a + hist_v[pl.ds(l * 256 + gbase, 16)]
        plsc.store_scatter(histc_v, [lane * 16 + (gbase >> 4)], a)


def _find_digit(histc_v, kk, lane):
    gtot = histc_v[pl.ds(0, 16)]
    for i in range(1, 16):
        gtot = gtot + histc_v[pl.ds(i * 16, 16)]
    grev = lax.rev(gtot, (0,))
    gcs = lax.cumsum(grev, axis=0)
    gidx = jnp.max(plsc.all_reduce_ffs(gcs >= kk))
    g_star = 15 - gidx
    above_g = jnp.sum(jnp.where(lane == gidx, gcs - grev, 0))

    h = plsc.load_gather(histc_v, [lane * 16 + g_star])
    rev = lax.rev(h, (0,))
    cs = lax.cumsum(rev, axis=0)
    idx = jnp.max(plsc.all_reduce_ffs((above_g + cs) >= kk))
    digit = g_star * 16 + 15 - idx
    above_d = above_g + jnp.sum(jnp.where(lane == idx, cs - rev, 0))
    return digit, kk - above_d


def _row_phase_a(row_v, candA_v, hist_v, histc_v, lane, laneoff,
                 lanecpl):
    ones_i = jnp.ones((16,), jnp.int32)
    mask0 = lane >= 1

    _zero_hist(hist_v)
    v0 = row_v[pl.ds(0, 16)]
    u0 = _ukey(v0)
    acc0 = jnp.where(mask0, jnp.exp(v0), jnp.float32(0.0))
    plsc.addupdate_scatter(
        hist_v, [_byte(u0, 24) + laneoff], ones_i, mask=mask0)

    @plsc.parallel_loop(16, COLS, step=16, unroll=8, carry=acc0)
    def _p1(i, acc):
        v = row_v[pl.ds(i, 16)]
        plsc.addupdate_scatter(
            hist_v, [_byte(_ukey(v), 24) + laneoff], ones_i)
        return acc + jnp.exp(v)

    total = jnp.sum(_p1)
    _collapse(hist_v, histc_v, lane)
    b1, rem = _find_digit(histc_v, jnp.int32(K), lane)

    ge0 = jnp.logical_and(_byte(u0, 24) >= b1, mask0)
    plsc.store_scatter(candA_v, [lanecpl], v0, mask=ge0)
    off0 = jnp.where(ge0, 1, 0)

    @plsc.parallel_loop(16, COLS, step=16, unroll=8, carry=off0)
    def _p2(i, off):
        v = row_v[pl.ds(i, 16)]
        ge = _byte(_ukey(v), 24) >= b1
        plsc.store_scatter(candA_v, [lanecpl + off], v, mask=ge)
        return off + jnp.where(ge, 1, 0)

    return total, b1, rem, _p2


def _row_phase_b(state, candA_v, candB_v, hist_v, histc_v, lane, laneoff,
                 lanecpl):
    total, b1, rem, n1p = state
    ones_i = jnp.ones((16,), jnp.int32)
    zf = jnp.zeros((16,), jnp.float32)
    zi = jnp.zeros((16,), jnp.int32)

    _zero_hist(hist_v)
    trips0 = jnp.max(n1p)

    def split(j, carry):
        s, off = carry
        v = plsc.load_gather(candA_v, [lanecpl + j])
        valid = j < n1p
        u = _ukey(v)
        hi = _byte(u, 24)
        gt = jnp.logical_and(valid, hi > b1)
        eq = jnp.logical_and(valid, hi == b1)
        s = s + jnp.where(gt, jnp.exp(v), jnp.float32(0.0))
        plsc.addupdate_scatter(hist_v, [_byte(u, 16) + laneoff], ones_i,
                               mask=eq)
        plsc.store_scatter(candB_v, [lanecpl + off], v, mask=eq)
        return s, off + jnp.where(eq, 1, 0)

    s_gt, n1 = lax.fori_loop(0, trips0, split, (zf, zi))
    _collapse(hist_v, histc_v, lane)
    b2, rem = _find_digit(histc_v, rem, lane)

    def cand_pass(src_v, dst_v, n_src, s_byte, bsel):
        _zero_hist(hist_v)
        trips = jnp.max(n_src)

        def body(j, carry):
            s, off = carry
            v = plsc.load_gather(src_v, [lanecpl + j])
            valid = j < n_src
            u = _ukey(v)
            d = _byte(u, s_byte)
            gt = jnp.logical_and(valid, d > bsel)
            eq = jnp.logical_and(valid, d == bsel)
            s = s + jnp.where(gt, jnp.exp(v), jnp.float32(0.0))
            plsc.addupdate_scatter(hist_v, [_byte(u, s_byte - 8) + laneoff],
                                   ones_i, mask=eq)
            plsc.store_scatter(dst_v, [lanecpl + off], v, mask=eq)
            return s, off + jnp.where(eq, 1, 0)

        s_add, n_dst = lax.fori_loop(0, trips, body, (zf, zi))
        _collapse(hist_v, histc_v, lane)
        return s_add, n_dst

    s3, n2 = cand_pass(candB_v, candA_v, n1, 16, b2)
    s_gt = s_gt + s3
    b3, rem = _find_digit(histc_v, rem, lane)

    s4, n3 = cand_pass(candA_v, candB_v, n2, 8, b3)
    s_gt = s_gt + s4
    b4, rem = _find_digit(histc_v, rem, lane)

    trips3 = jnp.max(n3)

    def p5(j, s):
        v = plsc.load_gather(candB_v, [lanecpl + j])
        valid = j < n3
        d = _byte(_ukey(v), 0)
        gt = jnp.logical_and(valid, d > b4)
        return s + jnp.where(gt, jnp.exp(v), jnp.float32(0.0))

    s_gt = s_gt + lax.fori_loop(0, trips3, p5, zf)
    sum_gt = jnp.sum(s_gt)

    t_u = jnp.bitwise_or(
        jnp.left_shift(
            jnp.bitwise_or(
                jnp.left_shift(
                    jnp.bitwise_or(jnp.left_shift(b1, 8), b2), 8),
                b3), 8),
        b4)
    tvec = jnp.full((16,), t_u, jnp.int32)
    bt = jnp.where(tvec < 0, tvec ^ jnp.int32(SIGN),
                   jnp.bitwise_xor(tvec, jnp.int32(-1)))
    exp_t = jnp.max(jnp.exp(plsc.bitcast(bt, jnp.float32)))

    top = sum_gt + rem.astype(jnp.float32) * exp_t
    return total - top


def _sc_body(y_hbm, out_hbm, row_v, candA_v, candB_v, hist_v, histc_v,
             ng_v, sem):
    wid = lax.axis_index("s") * NC + lax.axis_index("c")
    lane = lax.iota(jnp.int32, 16)
    laneoff = lane * 256
    lanecpl = lane * CPL

    ngvec0 = jnp.ones((16,), jnp.float32)
    pltpu.sync_copy(y_hbm.at[wid * RPW], row_v)

    def rbody(j, ngvec):
        state = _row_phase_a(row_v, candA_v, hist_v, histc_v, lane,
                             laneoff, lanecpl)
        nxt = wid * RPW + jnp.minimum(j + 1, RPW - 1)
        cp = pltpu.async_copy(y_hbm.at[nxt], row_v, sem)
        ng = _row_phase_b(state, candA_v, candB_v, hist_v, histc_v, lane,
                          laneoff, lanecpl)
        cp.wait()
        return jnp.where(lane == j, ng, ngvec)

    ngvec = lax.fori_loop(0, RPW, rbody, ngvec0)
    ng_v[...] = ngvec
    pltpu.sync_copy(ng_v, out_hbm.at[wid])


def _finish_kernel(ng_ref, y_ref, out_ref):
    ng = ng_ref[...]
    y = y_ref[...]
    col = lax.broadcasted_iota(jnp.int32, y.shape, 1)
    pos_sum = jnp.sum(jnp.where(col == 0, y, 0.0))
    out_ref[0, 0] = (jnp.sum(jnp.log(ng)) - pos_sum) * (1.0 / ROWS)


@jax.jit
def kernel(y_pred):
    mesh = plsc.VectorSubcoreMesh(core_axis_name="c", subcore_axis_name="s")
    sc = pl.kernel(
        _sc_body,
        out_type=jax.ShapeDtypeStruct((NW, 16), jnp.float32),
        mesh=mesh,
        compiler_params=pltpu.CompilerParams(needs_layout_passes=False),
        scratch_types=[
            pltpu.VMEM((COLS,), jnp.float32),
            pltpu.VMEM((COLS,), jnp.float32),
            pltpu.VMEM((COLS,), jnp.float32),
            pltpu.VMEM((16 * 256,), jnp.int32),
            pltpu.VMEM((256,), jnp.int32),
            pltpu.VMEM((16,), jnp.float32),
            pltpu.SemaphoreType.DMA,
        ],
    )
    ng = sc(y_pred)

    out = pl.pallas_call(
        _finish_kernel,
        grid=(1,),
        in_specs=[
            pl.BlockSpec((NW, 16), lambda i: (0, 0)),
            pl.BlockSpec((ROWS, 128), lambda i: (0, 0)),
        ],
        out_specs=pl.BlockSpec((1, 1), lambda i: (0, 0),
                               memory_space=pltpu.SMEM),
        out_shape=jax.ShapeDtypeStruct((1, 1), jnp.float32),
    )(ng, y_pred)
    return out[0, 0]
